# baseline (speedup 1.0000x reference)
import jax
import jax.numpy as jnp
from jax.experimental import pallas as pl

_TOKEN_BLOCK = 4096


def _router_block(x_ref, w_ref, probs_ref, map_ref):
    x = x_ref[...]
    w = w_ref[...]
    logits = jnp.dot(x, w, preferred_element_type=jnp.float32)
    m1 = jnp.max(logits, axis=-1, keepdims=True)
    lm = jnp.where(logits == m1, -jnp.inf, logits)
    m2 = jnp.max(lm, axis=-1, keepdims=True)
    rmap = logits >= m2
    e = jnp.exp(logits - m1)
    ones = jnp.ones((e.shape[1], 1), jnp.float32)
    s = jnp.dot(e, ones, preferred_element_type=jnp.float32)
    probs_ref[...] = jnp.where(rmap, e, 0.0) / s
    map_ref[...] = rmap


def kernel(hidden_states, router_weight):
    tokens, d_model = hidden_states.shape
    num_experts = router_weight.shape[1]
    block = _TOKEN_BLOCK
    grid = (tokens // block,)
    probs, routing_map = pl.pallas_call(
        _router_block,
        grid=grid,
        in_specs=[
            pl.BlockSpec((block, d_model), lambda i: (i, 0)),
            pl.BlockSpec((d_model, num_experts), lambda i: (0, 0)),
        ],
        out_specs=[
            pl.BlockSpec((block, num_experts), lambda i: (i, 0)),
            pl.BlockSpec((block, num_experts), lambda i: (i, 0)),
        ],
        out_shape=[
            jax.ShapeDtypeStruct((tokens, num_experts), jnp.float32),
            jax.ShapeDtypeStruct((tokens, num_experts), jnp.bool_),
        ],
    )(hidden_states, router_weight)
    return probs, routing_map


# speedup vs baseline: 1.1960x; 1.1960x over previous
import jax
import jax.numpy as jnp
from jax.experimental import pallas as pl

_TOKEN_BLOCK = 4096


def _router_block(x_ref, w_ref, probs_ref, map_ref):
    x = x_ref[...]
    w = w_ref[...]
    logits = jnp.dot(x, w, preferred_element_type=jnp.float32)
    m1 = jnp.max(logits, axis=-1, keepdims=True)
    lm = jnp.where(logits == m1, -jnp.inf, logits)
    m2 = jnp.max(lm, axis=-1, keepdims=True)
    rmap = logits >= m2
    e = jnp.exp(logits - m1)
    s = jnp.sum(e, axis=-1, keepdims=True)
    probs_ref[...] = jnp.where(rmap, e, 0.0) / s
    map_ref[...] = rmap


def kernel(hidden_states, router_weight):
    tokens, d_model = hidden_states.shape
    num_experts = router_weight.shape[1]
    block = _TOKEN_BLOCK
    grid = (tokens // block,)
    probs, routing_map = pl.pallas_call(
        _router_block,
        grid=grid,
        in_specs=[
            pl.BlockSpec((block, d_model), lambda i: (i, 0)),
            pl.BlockSpec((d_model, num_experts), lambda i: (0, 0)),
        ],
        out_specs=[
            pl.BlockSpec((block, num_experts), lambda i: (i, 0)),
            pl.BlockSpec((block, num_experts), lambda i: (i, 0)),
        ],
        out_shape=[
            jax.ShapeDtypeStruct((tokens, num_experts), jnp.float32),
            jax.ShapeDtypeStruct((tokens, num_experts), jnp.bool_),
        ],
    )(hidden_states, router_weight)
    return probs, routing_map
